# Initial kernel scaffold; baseline (speedup 1.0000x reference)
import jax
import jax.numpy as jnp
from jax.experimental import pallas as pl


def kernel(features, edges, pos_edges, W1, b1, Wf1, bf1, Wf2, bf2, Wc1, bc1, Wmu, bmu, Wls, bls, Wclf, bclf):
    raise NotImplementedError("write your pallas kernel here")


# speedup vs baseline: 315.4491x; 315.4491x over previous
import functools

import jax
import jax.numpy as jnp
from jax import lax
from jax.experimental import pallas as pl
from jax.experimental.pallas import tpu as pltpu
from jax.experimental.pallas import tpu_sc as plsc

G = 32
N = 2048
E = 16384
D = 128
PE = 128
NC, NS = 2, 16
LANES = 16


def _sc_edge_weights(edges):
    mesh = plsc.VectorSubcoreMesh(core_axis_name="c", subcore_axis_name="s")

    @functools.partial(
        pl.kernel,
        out_type=jax.ShapeDtypeStruct((G, N), jnp.float32),
        mesh=mesh,
        compiler_params=pltpu.CompilerParams(needs_layout_passes=False),
        scratch_types=[
            pltpu.VMEM((2, E), jnp.int32),
            pltpu.VMEM((LANES * N,), jnp.float32),
            pltpu.VMEM((N,), jnp.float32),
            pltpu.VMEM((N,), jnp.float32),
        ],
    )
    def k(edges_hbm, w_hbm, edges_v, hist_v, dinv_v, w_v):
        g = lax.axis_index("s") * NC + lax.axis_index("c")
        pltpu.sync_copy(edges_hbm.at[g], edges_v)
        lane_off = lax.iota(jnp.int32, 16) * N
        zeros16 = jnp.zeros((16,), jnp.float32)
        ones16 = jnp.ones((16,), jnp.float32)

        def zero_body(j, carry):
            for l in range(LANES):
                hist_v[pl.ds(l * N + j * 16, 16)] = zeros16
            return carry

        lax.fori_loop(0, N // 16, zero_body, 0)

        def hist_body(i, carry):
            c = edges_v[1, pl.ds(i * 16, 16)]
            plsc.addupdate_scatter(hist_v, [lane_off + c], ones16)
            return carry

        lax.fori_loop(0, E // 16, hist_body, 0)

        def dinv_body(j, carry):
            acc = hist_v[pl.ds(j * 16, 16)]
            for l in range(1, LANES):
                acc = acc + hist_v[pl.ds(l * N + j * 16, 16)]
            deg = acc + 1.0
            bits = plsc.bitcast(deg, jnp.int32)
            y = plsc.bitcast(jnp.int32(0x5F3759DF) - lax.shift_right_logical(bits, 1),
                             jnp.float32)
            y = y * (1.5 - 0.5 * deg * y * y)
            y = y * (1.5 - 0.5 * deg * y * y)
            y = y * (1.5 - 0.5 * deg * y * y)
            dinv_v[pl.ds(j * 16, 16)] = y
            for l in range(LANES):
                hist_v[pl.ds(l * N + j * 16, 16)] = zeros16
            return carry

        lax.fori_loop(0, N // 16, dinv_body, 0)

        def s_body(i, carry):
            r = edges_v[0, pl.ds(i * 16, 16)]
            c = edges_v[1, pl.ds(i * 16, 16)]
            dc = plsc.load_gather(dinv_v, [c])
            plsc.addupdate_scatter(hist_v, [lane_off + r], dc)
            return carry

        lax.fori_loop(0, E // 16, s_body, 0)

        def w_body(j, carry):
            acc = hist_v[pl.ds(j * 16, 16)]
            for l in range(1, LANES):
                acc = acc + hist_v[pl.ds(l * N + j * 16, 16)]
            dv = dinv_v[pl.ds(j * 16, 16)]
            w_v[pl.ds(j * 16, 16)] = dv * (acc + dv)
            return carry

        lax.fori_loop(0, N // 16, w_body, 0)

        pltpu.sync_copy(w_v, w_hbm.at[g])

    return k(edges)


def _tc_head(w, features, pos_edges, W1, b1, Wc1, bc1, Wmu, bmu, Wclf, bclf):
    D1 = W1.shape[1]
    H = Wmu.shape[1]
    L = Wclf.shape[1]

    def body(w_ref, x_ref, pe_ref, W1_ref, b1_ref, Wc1_ref, bc1_ref,
             Wmu_ref, bmu_ref, Wclf_ref, bclf_ref, out_ref, v_acc):
        g = pl.program_id(0)
        xv = x_ref[0]
        wv = w_ref[0]
        v_acc[pl.ds(g, 1), :] = jnp.dot(wv, xv, preferred_element_type=jnp.float32)

        @pl.when(g == G - 1)
        def _():
            emb = (jnp.dot(v_acc[...], W1_ref[...],
                           preferred_element_type=jnp.float32)
                   + N * b1_ref[...]) * (1.0 / 16.0)
            pe = pe_ref[...]
            gids = lax.broadcasted_iota(jnp.int32, (PE, G), 1)
            oh_r = (pe[0][:, None] == gids).astype(jnp.float32)
            oh_c = (pe[1][:, None] == gids).astype(jnp.float32)
            A = lax.dot_general(oh_c, oh_r, (((0,), (0,)), ((), ())),
                                preferred_element_type=jnp.float32)
            eye = (lax.broadcasted_iota(jnp.int32, (G, G), 0)
                   == lax.broadcasted_iota(jnp.int32, (G, G), 1)
                   ).astype(jnp.float32)
            A1 = A + eye
            deg = jnp.sum(A1, axis=1)
            dinv = lax.rsqrt(deg)
            Ahat = dinv[:, None] * A1 * dinv[None, :]

            def mm(a, b):
                return jnp.dot(a, b, preferred_element_type=jnp.float32)

            h1 = jnp.maximum(mm(Ahat, mm(emb, Wc1_ref[...])) + bc1_ref[...], 0.0)
            mu = mm(Ahat, mm(h1, Wmu_ref[...])) + bmu_ref[...]
            logits = mm(mu, Wclf_ref[...]) + bclf_ref[...]
            m = jnp.max(logits, axis=1, keepdims=True)
            lse = m + jnp.log(jnp.sum(jnp.exp(logits - m), axis=1, keepdims=True))
            out_ref[...] = logits - lse

    full = lambda shape: pl.BlockSpec(shape, lambda g: tuple(0 for _ in shape))
    return pl.pallas_call(
        body,
        grid=(G,),
        in_specs=[
            pl.BlockSpec((1, 1, N), lambda g: (g, 0, 0)),
            pl.BlockSpec((1, N, D), lambda g: (g, 0, 0)),
            full((2, PE)),
            full((D, D1)),
            full((D1,)),
            full((D1, Wc1.shape[1])),
            full((Wc1.shape[1],)),
            full((2 * H, H)),
            full((H,)),
            full((H, L)),
            full((L,)),
        ],
        out_specs=full((G, L)),
        out_shape=jax.ShapeDtypeStruct((G, L), jnp.float32),
        scratch_shapes=[pltpu.VMEM((G, D1), jnp.float32)],
    )(w.reshape(G, 1, N), features, pos_edges, W1, b1, Wc1, bc1, Wmu, bmu,
      Wclf, bclf)


def kernel(features, edges, pos_edges, W1, b1, Wf1, bf1, Wf2, bf2,
           Wc1, bc1, Wmu, bmu, Wls, bls, Wclf, bclf):
    w = _sc_edge_weights(edges)
    return _tc_head(w, features, pos_edges, W1, b1, Wc1, bc1, Wmu, bmu,
                    Wclf, bclf)


# speedup vs baseline: 317.5228x; 1.0066x over previous
import functools

import jax
import jax.numpy as jnp
from jax import lax
from jax.experimental import pallas as pl
from jax.experimental.pallas import tpu as pltpu
from jax.experimental.pallas import tpu_sc as plsc

G = 32
N = 2048
E = 16384
D = 128
PE = 128
NC, NS = 2, 16
LANES = 16


def _sc_edge_weights(edges):
    mesh = plsc.VectorSubcoreMesh(core_axis_name="c", subcore_axis_name="s")

    @functools.partial(
        pl.kernel,
        out_type=jax.ShapeDtypeStruct((G, N), jnp.float32),
        mesh=mesh,
        compiler_params=pltpu.CompilerParams(needs_layout_passes=False),
        scratch_types=[
            pltpu.VMEM((2, E), jnp.int32),
            pltpu.VMEM((LANES * N,), jnp.float32),
            pltpu.VMEM((N,), jnp.float32),
            pltpu.VMEM((N,), jnp.float32),
        ],
    )
    def k(edges_hbm, w_hbm, edges_v, hist_v, dinv_v, w_v):
        g = lax.axis_index("s") * NC + lax.axis_index("c")
        pltpu.sync_copy(edges_hbm.at[g], edges_v)
        lane_off = lax.iota(jnp.int32, 16) * N
        zeros16 = jnp.zeros((16,), jnp.float32)
        ones16 = jnp.ones((16,), jnp.float32)

        def zero_body(j, carry):
            for l in range(LANES):
                hist_v[pl.ds(l * N + j * 16, 16)] = zeros16
            return carry

        lax.fori_loop(0, N // 16, zero_body, 0)

        def hist_body(i, carry):
            for u in range(8):
                c = edges_v[1, pl.ds(i * 128 + u * 16, 16)]
                plsc.addupdate_scatter(hist_v, [lane_off + c], ones16)
            return carry

        lax.fori_loop(0, E // 128, hist_body, 0)

        def dinv_body(j, carry):
            acc = hist_v[pl.ds(j * 16, 16)]
            for l in range(1, LANES):
                acc = acc + hist_v[pl.ds(l * N + j * 16, 16)]
            deg = acc + 1.0
            bits = plsc.bitcast(deg, jnp.int32)
            y = plsc.bitcast(jnp.int32(0x5F3759DF) - lax.shift_right_logical(bits, 1),
                             jnp.float32)
            y = y * (1.5 - 0.5 * deg * y * y)
            y = y * (1.5 - 0.5 * deg * y * y)
            y = y * (1.5 - 0.5 * deg * y * y)
            dinv_v[pl.ds(j * 16, 16)] = y
            for l in range(LANES):
                hist_v[pl.ds(l * N + j * 16, 16)] = zeros16
            return carry

        lax.fori_loop(0, N // 16, dinv_body, 0)

        def s_body(i, carry):
            for u in range(8):
                sl = pl.ds(i * 128 + u * 16, 16)
                r = edges_v[0, sl]
                c = edges_v[1, sl]
                dc = plsc.load_gather(dinv_v, [c])
                plsc.addupdate_scatter(hist_v, [lane_off + r], dc)
            return carry

        lax.fori_loop(0, E // 128, s_body, 0)

        def w_body(j, carry):
            acc = hist_v[pl.ds(j * 16, 16)]
            for l in range(1, LANES):
                acc = acc + hist_v[pl.ds(l * N + j * 16, 16)]
            dv = dinv_v[pl.ds(j * 16, 16)]
            w_v[pl.ds(j * 16, 16)] = dv * (acc + dv)
            return carry

        lax.fori_loop(0, N // 16, w_body, 0)

        pltpu.sync_copy(w_v, w_hbm.at[g])

    return k(edges)


def _tc_head(w, features, pos_edges, W1, b1, Wc1, bc1, Wmu, bmu, Wclf, bclf):
    D1 = W1.shape[1]
    H = Wmu.shape[1]
    L = Wclf.shape[1]

    def body(w_ref, x_ref, pe_ref, W1_ref, b1_ref, Wc1_ref, bc1_ref,
             Wmu_ref, bmu_ref, Wclf_ref, bclf_ref, out_ref, v_acc):
        g = pl.program_id(0)

        @pl.when(g == 0)
        def _():
            v_acc[...] = jnp.zeros((G, D1), jnp.float32)

        xv = x_ref[0]
        wv = w_ref[0]
        rows = lax.broadcasted_iota(jnp.int32, (G, N), 0)
        P = jnp.where(rows == g, jnp.broadcast_to(wv, (G, N)), 0.0)
        v_acc[...] += jnp.dot(P, xv, preferred_element_type=jnp.float32)

        @pl.when(g == G - 1)
        def _():
            emb = (jnp.dot(v_acc[...], W1_ref[...],
                           preferred_element_type=jnp.float32)
                   + N * b1_ref[...]) * (1.0 / 16.0)
            pe = pe_ref[...]
            gids = lax.broadcasted_iota(jnp.int32, (PE, G), 1)
            oh_r = (pe[0][:, None] == gids).astype(jnp.float32)
            oh_c = (pe[1][:, None] == gids).astype(jnp.float32)
            A = lax.dot_general(oh_c, oh_r, (((0,), (0,)), ((), ())),
                                preferred_element_type=jnp.float32)
            eye = (lax.broadcasted_iota(jnp.int32, (G, G), 0)
                   == lax.broadcasted_iota(jnp.int32, (G, G), 1)
                   ).astype(jnp.float32)
            A1 = A + eye
            deg = jnp.sum(A1, axis=1)
            dinv = lax.rsqrt(deg)
            Ahat = dinv[:, None] * A1 * dinv[None, :]

            def mm(a, b):
                return jnp.dot(a, b, preferred_element_type=jnp.float32)

            h1 = jnp.maximum(mm(Ahat, mm(emb, Wc1_ref[...])) + bc1_ref[...], 0.0)
            mu = mm(Ahat, mm(h1, Wmu_ref[...])) + bmu_ref[...]
            logits = mm(mu, Wclf_ref[...]) + bclf_ref[...]
            m = jnp.max(logits, axis=1, keepdims=True)
            lse = m + jnp.log(jnp.sum(jnp.exp(logits - m), axis=1, keepdims=True))
            out_ref[...] = logits - lse

    full = lambda shape: pl.BlockSpec(shape, lambda g: tuple(0 for _ in shape))
    return pl.pallas_call(
        body,
        grid=(G,),
        in_specs=[
            pl.BlockSpec((1, 1, N), lambda g: (g, 0, 0)),
            pl.BlockSpec((1, N, D), lambda g: (g, 0, 0)),
            full((2, PE)),
            full((D, D1)),
            full((D1,)),
            full((D1, Wc1.shape[1])),
            full((Wc1.shape[1],)),
            full((2 * H, H)),
            full((H,)),
            full((H, L)),
            full((L,)),
        ],
        out_specs=full((G, L)),
        out_shape=jax.ShapeDtypeStruct((G, L), jnp.float32),
        scratch_shapes=[pltpu.VMEM((G, D1), jnp.float32)],
    )(w.reshape(G, 1, N), features, pos_edges, W1, b1, Wc1, bc1, Wmu, bmu,
      Wclf, bclf)


def kernel(features, edges, pos_edges, W1, b1, Wf1, bf1, Wf2, bf2,
           Wc1, bc1, Wmu, bmu, Wls, bls, Wclf, bclf):
    w = _sc_edge_weights(edges)
    return _tc_head(w, features, pos_edges, W1, b1, Wc1, bc1, Wmu, bmu,
                    Wclf, bclf)


# speedup vs baseline: 392.3021x; 1.2355x over previous
import functools

import jax
import jax.numpy as jnp
from jax import lax
from jax.experimental import pallas as pl
from jax.experimental.pallas import tpu as pltpu
from jax.experimental.pallas import tpu_sc as plsc

G = 32
N = 2048
E = 16384
D = 128
PE = 128
NC, NS = 2, 16
LANES = 16


def _sc_edge_weights(edges):
    mesh = plsc.VectorSubcoreMesh(core_axis_name="c", subcore_axis_name="s")

    @functools.partial(
        pl.kernel,
        out_type=jax.ShapeDtypeStruct((G, N), jnp.float32),
        mesh=mesh,
        compiler_params=pltpu.CompilerParams(needs_layout_passes=False),
        scratch_types=[
            pltpu.VMEM((2, E), jnp.int32),
            pltpu.VMEM((LANES * N,), jnp.float32),
            pltpu.VMEM((N,), jnp.float32),
            pltpu.VMEM((N,), jnp.float32),
        ],
    )
    def k(edges_hbm, w_hbm, edges_v, hist_v, dinv_v, w_v):
        g = lax.axis_index("s") * NC + lax.axis_index("c")
        pltpu.sync_copy(edges_hbm.at[g], edges_v)
        lane_off = lax.iota(jnp.int32, 16) * N
        zeros16 = jnp.zeros((16,), jnp.float32)
        ones16 = jnp.ones((16,), jnp.float32)

        def zero_body(j, carry):
            for l in range(LANES):
                hist_v[pl.ds(l * N + j * 16, 16)] = zeros16
            return carry

        lax.fori_loop(0, N // 16, zero_body, 0)

        def hist_body(i, carry):
            for u in range(8):
                c = edges_v[1, pl.ds(i * 128 + u * 16, 16)]
                plsc.addupdate_scatter(hist_v, [lane_off + c], ones16)
            return carry

        lax.fori_loop(0, E // 128, hist_body, 0)

        def dinv_body(j, carry):
            acc = hist_v[pl.ds(j * 16, 16)]
            for l in range(1, LANES):
                acc = acc + hist_v[pl.ds(l * N + j * 16, 16)]
            deg = acc + 1.0
            bits = plsc.bitcast(deg, jnp.int32)
            y = plsc.bitcast(jnp.int32(0x5F3759DF) - lax.shift_right_logical(bits, 1),
                             jnp.float32)
            y = y * (1.5 - 0.5 * deg * y * y)
            y = y * (1.5 - 0.5 * deg * y * y)
            y = y * (1.5 - 0.5 * deg * y * y)
            dinv_v[pl.ds(j * 16, 16)] = y
            for l in range(LANES):
                hist_v[pl.ds(l * N + j * 16, 16)] = zeros16
            return carry

        lax.fori_loop(0, N // 16, dinv_body, 0)

        def s_body(i, carry):
            for u in range(8):
                sl = pl.ds(i * 128 + u * 16, 16)
                r = edges_v[0, sl]
                c = edges_v[1, sl]
                dc = plsc.load_gather(dinv_v, [c])
                plsc.addupdate_scatter(hist_v, [lane_off + r], dc)
            return carry

        lax.fori_loop(0, E // 128, s_body, 0)

        def w_body(j, carry):
            acc = hist_v[pl.ds(j * 16, 16)]
            for l in range(1, LANES):
                acc = acc + hist_v[pl.ds(l * N + j * 16, 16)]
            dv = dinv_v[pl.ds(j * 16, 16)]
            w_v[pl.ds(j * 16, 16)] = dv * (acc + dv)
            return carry

        lax.fori_loop(0, N // 16, w_body, 0)

        pltpu.sync_copy(w_v, w_hbm.at[g])

    return k(edges)


def _tc_head(w, features, pos_edges, W1, b1, Wc1, bc1, Wmu, bmu, Wclf, bclf):
    D1 = W1.shape[1]
    H = Wmu.shape[1]
    L = Wclf.shape[1]

    B = 4

    def body(w_ref, x_ref, pe_ref, W1_ref, b1_ref, Wc1_ref, bc1_ref,
             Wmu_ref, bmu_ref, Wclf_ref, bclf_ref, out_ref, v_acc):
        g = pl.program_id(0)

        @pl.when(g == 0)
        def _():
            v_acc[...] = jnp.zeros((G, D1), jnp.float32)

        wv = w_ref[...]
        rows = lax.broadcasted_iota(jnp.int32, (G, N), 0)
        acc = jnp.zeros((G, D1), jnp.float32)
        for b in range(B):
            gg = g * B + b
            P = jnp.where(rows == gg, wv, 0.0)
            acc = acc + jnp.dot(P, x_ref[b], preferred_element_type=jnp.float32)
        v_acc[...] += acc

        @pl.when(g == G // B - 1)
        def _():
            emb = (jnp.dot(v_acc[...], W1_ref[...],
                           preferred_element_type=jnp.float32)
                   + N * b1_ref[...]) * (1.0 / 16.0)
            pe = pe_ref[...]
            gids = lax.broadcasted_iota(jnp.int32, (PE, G), 1)
            oh_r = (pe[0][:, None] == gids).astype(jnp.float32)
            oh_c = (pe[1][:, None] == gids).astype(jnp.float32)
            A = lax.dot_general(oh_c, oh_r, (((0,), (0,)), ((), ())),
                                preferred_element_type=jnp.float32)
            eye = (lax.broadcasted_iota(jnp.int32, (G, G), 0)
                   == lax.broadcasted_iota(jnp.int32, (G, G), 1)
                   ).astype(jnp.float32)
            A1 = A + eye
            deg = jnp.sum(A1, axis=1)
            dinv = lax.rsqrt(deg)
            Ahat = dinv[:, None] * A1 * dinv[None, :]

            def mm(a, b):
                return jnp.dot(a, b, preferred_element_type=jnp.float32)

            h1 = jnp.maximum(mm(Ahat, mm(emb, Wc1_ref[...])) + bc1_ref[...], 0.0)
            mu = mm(Ahat, mm(h1, Wmu_ref[...])) + bmu_ref[...]
            logits = mm(mu, Wclf_ref[...]) + bclf_ref[...]
            m = jnp.max(logits, axis=1, keepdims=True)
            lse = m + jnp.log(jnp.sum(jnp.exp(logits - m), axis=1, keepdims=True))
            out_ref[...] = logits - lse

    full = lambda shape: pl.BlockSpec(shape, lambda g: tuple(0 for _ in shape))
    return pl.pallas_call(
        body,
        grid=(G // B,),
        in_specs=[
            full((G, N)),
            pl.BlockSpec((B, N, D), lambda g: (g, 0, 0)),
            full((2, PE)),
            full((D, D1)),
            full((D1,)),
            full((D1, Wc1.shape[1])),
            full((Wc1.shape[1],)),
            full((2 * H, H)),
            full((H,)),
            full((H, L)),
            full((L,)),
        ],
        out_specs=full((G, L)),
        out_shape=jax.ShapeDtypeStruct((G, L), jnp.float32),
        scratch_shapes=[pltpu.VMEM((G, D1), jnp.float32)],
    )(w, features, pos_edges, W1, b1, Wc1, bc1, Wmu, bmu, Wclf, bclf)


def kernel(features, edges, pos_edges, W1, b1, Wf1, bf1, Wf2, bf2,
           Wc1, bc1, Wmu, bmu, Wls, bls, Wclf, bclf):
    w = _sc_edge_weights(edges)
    return _tc_head(w, features, pos_edges, W1, b1, Wc1, bc1, Wmu, bmu,
                    Wclf, bclf)


# speedup vs baseline: 408.1019x; 1.0403x over previous
import functools

import jax
import jax.numpy as jnp
from jax import lax
from jax.experimental import pallas as pl
from jax.experimental.pallas import tpu as pltpu
from jax.experimental.pallas import tpu_sc as plsc

G = 32
N = 2048
E = 16384
D = 128
PE = 128
NC, NS = 2, 16
LANES = 16


def _sc_edge_weights(edges):
    mesh = plsc.VectorSubcoreMesh(core_axis_name="c", subcore_axis_name="s")

    @functools.partial(
        pl.kernel,
        out_type=jax.ShapeDtypeStruct((G, N), jnp.float32),
        mesh=mesh,
        compiler_params=pltpu.CompilerParams(needs_layout_passes=False),
        scratch_types=[
            pltpu.VMEM((2, E), jnp.int32),
            pltpu.VMEM((LANES * N,), jnp.float32),
            pltpu.VMEM((N,), jnp.float32),
            pltpu.VMEM((N,), jnp.float32),
            pltpu.VMEM((N,), jnp.float32),
            pltpu.SemaphoreType.DMA,
        ],
    )
    def k(edges_hbm, w_hbm, edges_v, hist_v, dinv_v, cnt_v, w_v, sem):
        g = lax.axis_index("s") * NC + lax.axis_index("c")
        cp = pltpu.async_copy(edges_hbm.at[g], edges_v, sem)
        lane_off = lax.iota(jnp.int32, 16) * N
        zeros16 = jnp.zeros((16,), jnp.float32)
        ones16 = jnp.ones((16,), jnp.float32)

        def zero_body(j, carry):
            for l in range(LANES):
                hist_v[pl.ds(l * N + j * 16, 16)] = zeros16
            return carry

        lax.fori_loop(0, N // 16, zero_body, 0)
        cp.wait()

        def hist_body(i, carry):
            for u in range(8):
                c = edges_v[1, pl.ds(i * 128 + u * 16, 16)]
                plsc.addupdate_scatter(hist_v, [lane_off + c], ones16)
            return carry

        lax.fori_loop(0, E // 128, hist_body, 0)

        def dinv_body(j, carry):
            acc = hist_v[pl.ds(j * 16, 16)]
            for l in range(1, LANES):
                acc = acc + hist_v[pl.ds(l * N + j * 16, 16)]
            cnt_v[pl.ds(j * 16, 16)] = acc
            deg = acc + 1.0
            bits = plsc.bitcast(deg, jnp.int32)
            y = plsc.bitcast(jnp.int32(0x5F3759DF) - lax.shift_right_logical(bits, 1),
                             jnp.float32)
            y = y * (1.5 - 0.5 * deg * y * y)
            y = y * (1.5 - 0.5 * deg * y * y)
            dinv_v[pl.ds(j * 16, 16)] = y
            return carry

        lax.fori_loop(0, N // 16, dinv_body, 0)

        def s_body(i, carry):
            for u in range(8):
                sl = pl.ds(i * 128 + u * 16, 16)
                r = edges_v[0, sl]
                c = edges_v[1, sl]
                dc = plsc.load_gather(dinv_v, [c])
                plsc.addupdate_scatter(hist_v, [lane_off + r], dc)
            return carry

        lax.fori_loop(0, E // 128, s_body, 0)

        def w_body(j, carry):
            acc = hist_v[pl.ds(j * 16, 16)]
            for l in range(1, LANES):
                acc = acc + hist_v[pl.ds(l * N + j * 16, 16)]
            s = acc - cnt_v[pl.ds(j * 16, 16)]
            dv = dinv_v[pl.ds(j * 16, 16)]
            w_v[pl.ds(j * 16, 16)] = dv * (s + dv)
            return carry

        lax.fori_loop(0, N // 16, w_body, 0)

        pltpu.sync_copy(w_v, w_hbm.at[g])

    return k(edges)


def _tc_head(w, features, pos_edges, W1, b1, Wc1, bc1, Wmu, bmu, Wclf, bclf):
    D1 = W1.shape[1]
    H = Wmu.shape[1]
    L = Wclf.shape[1]

    B = 4

    def body(w_ref, x_ref, pe_ref, W1_ref, b1_ref, Wc1_ref, bc1_ref,
             Wmu_ref, bmu_ref, Wclf_ref, bclf_ref, out_ref, v_acc):
        g = pl.program_id(0)

        @pl.when(g == 0)
        def _():
            v_acc[...] = jnp.zeros((G, D1), jnp.float32)

        wv = w_ref[...]
        rows = lax.broadcasted_iota(jnp.int32, (G, N), 0)
        acc = jnp.zeros((G, D1), jnp.float32)
        for b in range(B):
            gg = g * B + b
            P = jnp.where(rows == gg, wv, 0.0)
            acc = acc + jnp.dot(P, x_ref[b], preferred_element_type=jnp.float32)
        v_acc[...] += acc

        @pl.when(g == G // B - 1)
        def _():
            emb = (jnp.dot(v_acc[...], W1_ref[...],
                           preferred_element_type=jnp.float32)
                   + N * b1_ref[...]) * (1.0 / 16.0)
            pe = pe_ref[...]
            gids = lax.broadcasted_iota(jnp.int32, (PE, G), 1)
            oh_r = (pe[0][:, None] == gids).astype(jnp.float32)
            oh_c = (pe[1][:, None] == gids).astype(jnp.float32)
            A = lax.dot_general(oh_c, oh_r, (((0,), (0,)), ((), ())),
                                preferred_element_type=jnp.float32)
            eye = (lax.broadcasted_iota(jnp.int32, (G, G), 0)
                   == lax.broadcasted_iota(jnp.int32, (G, G), 1)
                   ).astype(jnp.float32)
            A1 = A + eye
            deg = jnp.sum(A1, axis=1)
            dinv = lax.rsqrt(deg)
            Ahat = dinv[:, None] * A1 * dinv[None, :]

            def mm(a, b):
                return jnp.dot(a, b, preferred_element_type=jnp.float32)

            h1 = jnp.maximum(mm(Ahat, mm(emb, Wc1_ref[...])) + bc1_ref[...], 0.0)
            mu = mm(Ahat, mm(h1, Wmu_ref[...])) + bmu_ref[...]
            logits = mm(mu, Wclf_ref[...]) + bclf_ref[...]
            m = jnp.max(logits, axis=1, keepdims=True)
            lse = m + jnp.log(jnp.sum(jnp.exp(logits - m), axis=1, keepdims=True))
            out_ref[...] = logits - lse

    full = lambda shape: pl.BlockSpec(shape, lambda g: tuple(0 for _ in shape))
    return pl.pallas_call(
        body,
        grid=(G // B,),
        in_specs=[
            full((G, N)),
            pl.BlockSpec((B, N, D), lambda g: (g, 0, 0)),
            full((2, PE)),
            full((D, D1)),
            full((D1,)),
            full((D1, Wc1.shape[1])),
            full((Wc1.shape[1],)),
            full((2 * H, H)),
            full((H,)),
            full((H, L)),
            full((L,)),
        ],
        out_specs=full((G, L)),
        out_shape=jax.ShapeDtypeStruct((G, L), jnp.float32),
        scratch_shapes=[pltpu.VMEM((G, D1), jnp.float32)],
    )(w, features, pos_edges, W1, b1, Wc1, bc1, Wmu, bmu, Wclf, bclf)


def kernel(features, edges, pos_edges, W1, b1, Wf1, bf1, Wf2, bf2,
           Wc1, bc1, Wmu, bmu, Wls, bls, Wclf, bclf):
    w = _sc_edge_weights(edges)
    return _tc_head(w, features, pos_edges, W1, b1, Wc1, bc1, Wmu, bmu,
                    Wclf, bclf)


# speedup vs baseline: 417.2819x; 1.0225x over previous
import functools

import jax
import jax.numpy as jnp
from jax import lax
from jax.experimental import pallas as pl
from jax.experimental.pallas import tpu as pltpu
from jax.experimental.pallas import tpu_sc as plsc

G = 32
N = 2048
E = 16384
D = 128
PE = 128
NC, NS = 2, 16
LANES = 16


def _sc_edge_weights(edges):
    mesh = plsc.VectorSubcoreMesh(core_axis_name="c", subcore_axis_name="s")

    @functools.partial(
        pl.kernel,
        out_type=jax.ShapeDtypeStruct((G, N), jnp.float32),
        mesh=mesh,
        compiler_params=pltpu.CompilerParams(needs_layout_passes=False),
        scratch_types=[
            pltpu.VMEM((2, E), jnp.int32),
            pltpu.VMEM((LANES * N,), jnp.float32),
            pltpu.VMEM((LANES * N,), jnp.float32),
            pltpu.VMEM((N,), jnp.float32),
            pltpu.VMEM((N,), jnp.float32),
            pltpu.VMEM((N,), jnp.float32),
            pltpu.SemaphoreType.DMA,
        ],
    )
    def k(edges_hbm, w_hbm, edges_v, hist_v, hist2_v, dinv_v, cnt_v, w_v, sem):
        g = lax.axis_index("s") * NC + lax.axis_index("c")
        cp = pltpu.async_copy(edges_hbm.at[g], edges_v, sem)
        lane_off = lax.iota(jnp.int32, 16) * N
        zeros16 = jnp.zeros((16,), jnp.float32)
        ones16 = jnp.ones((16,), jnp.float32)

        def zero_body(j, carry):
            for l in range(LANES):
                hist_v[pl.ds(l * N + j * 16, 16)] = zeros16
                hist2_v[pl.ds(l * N + j * 16, 16)] = zeros16
            return carry

        lax.fori_loop(0, N // 16, zero_body, 0)
        cp.wait()

        def hist_body(i, carry):
            for u in range(8):
                c = edges_v[1, pl.ds(i * 128 + u * 16, 16)]
                plsc.addupdate_scatter(hist_v if u % 2 == 0 else hist2_v,
                                       [lane_off + c], ones16)
            return carry

        lax.fori_loop(0, E // 128, hist_body, 0)

        def dinv_body(j, carry):
            acc = hist_v[pl.ds(j * 16, 16)]
            acc2 = hist2_v[pl.ds(j * 16, 16)]
            for l in range(1, LANES):
                acc = acc + hist_v[pl.ds(l * N + j * 16, 16)]
                acc2 = acc2 + hist2_v[pl.ds(l * N + j * 16, 16)]
            acc = acc + acc2
            cnt_v[pl.ds(j * 16, 16)] = acc
            deg = acc + 1.0
            bits = plsc.bitcast(deg, jnp.int32)
            y = plsc.bitcast(jnp.int32(0x5F3759DF) - lax.shift_right_logical(bits, 1),
                             jnp.float32)
            y = y * (1.5 - 0.5 * deg * y * y)
            y = y * (1.5 - 0.5 * deg * y * y)
            dinv_v[pl.ds(j * 16, 16)] = y
            return carry

        lax.fori_loop(0, N // 16, dinv_body, 0)

        def s_body(i, carry):
            for u in range(8):
                sl = pl.ds(i * 128 + u * 16, 16)
                r = edges_v[0, sl]
                c = edges_v[1, sl]
                dc = plsc.load_gather(dinv_v, [c])
                plsc.addupdate_scatter(hist_v if u % 2 == 0 else hist2_v,
                                       [lane_off + r], dc)
            return carry

        lax.fori_loop(0, E // 128, s_body, 0)

        def w_body(j, carry):
            acc = hist_v[pl.ds(j * 16, 16)]
            acc2 = hist2_v[pl.ds(j * 16, 16)]
            for l in range(1, LANES):
                acc = acc + hist_v[pl.ds(l * N + j * 16, 16)]
                acc2 = acc2 + hist2_v[pl.ds(l * N + j * 16, 16)]
            s = acc + acc2 - cnt_v[pl.ds(j * 16, 16)]
            dv = dinv_v[pl.ds(j * 16, 16)]
            w_v[pl.ds(j * 16, 16)] = dv * (s + dv)
            return carry

        lax.fori_loop(0, N // 16, w_body, 0)

        pltpu.sync_copy(w_v, w_hbm.at[g])

    return k(edges)


def _tc_head(w, features, pos_edges, W1, b1, Wc1, bc1, Wmu, bmu, Wclf, bclf):
    D1 = W1.shape[1]
    H = Wmu.shape[1]
    L = Wclf.shape[1]

    B = 4

    def body(w_ref, x_ref, pe_ref, W1_ref, b1_ref, Wc1_ref, bc1_ref,
             Wmu_ref, bmu_ref, Wclf_ref, bclf_ref, out_ref, v_acc):
        g = pl.program_id(0)

        @pl.when(g == 0)
        def _():
            v_acc[...] = jnp.zeros((G, D1), jnp.float32)

        wv = w_ref[...]
        rows = lax.broadcasted_iota(jnp.int32, (G, N), 0)
        acc = jnp.zeros((G, D1), jnp.float32)
        for b in range(B):
            gg = g * B + b
            P = jnp.where(rows == gg, wv, 0.0)
            acc = acc + jnp.dot(P, x_ref[b], preferred_element_type=jnp.float32)
        v_acc[...] += acc

        @pl.when(g == G // B - 1)
        def _():
            emb = (jnp.dot(v_acc[...], W1_ref[...],
                           preferred_element_type=jnp.float32)
                   + N * b1_ref[...]) * (1.0 / 16.0)
            pe = pe_ref[...]
            gids = lax.broadcasted_iota(jnp.int32, (PE, G), 1)
            oh_r = (pe[0][:, None] == gids).astype(jnp.float32)
            oh_c = (pe[1][:, None] == gids).astype(jnp.float32)
            A = lax.dot_general(oh_c, oh_r, (((0,), (0,)), ((), ())),
                                preferred_element_type=jnp.float32)
            eye = (lax.broadcasted_iota(jnp.int32, (G, G), 0)
                   == lax.broadcasted_iota(jnp.int32, (G, G), 1)
                   ).astype(jnp.float32)
            A1 = A + eye
            deg = jnp.sum(A1, axis=1)
            dinv = lax.rsqrt(deg)
            Ahat = dinv[:, None] * A1 * dinv[None, :]

            def mm(a, b):
                return jnp.dot(a, b, preferred_element_type=jnp.float32)

            h1 = jnp.maximum(mm(Ahat, mm(emb, Wc1_ref[...])) + bc1_ref[...], 0.0)
            mu = mm(Ahat, mm(h1, Wmu_ref[...])) + bmu_ref[...]
            logits = mm(mu, Wclf_ref[...]) + bclf_ref[...]
            m = jnp.max(logits, axis=1, keepdims=True)
            lse = m + jnp.log(jnp.sum(jnp.exp(logits - m), axis=1, keepdims=True))
            out_ref[...] = logits - lse

    full = lambda shape: pl.BlockSpec(shape, lambda g: tuple(0 for _ in shape))
    return pl.pallas_call(
        body,
        grid=(G // B,),
        in_specs=[
            full((G, N)),
            pl.BlockSpec((B, N, D), lambda g: (g, 0, 0)),
            full((2, PE)),
            full((D, D1)),
            full((D1,)),
            full((D1, Wc1.shape[1])),
            full((Wc1.shape[1],)),
            full((2 * H, H)),
            full((H,)),
            full((H, L)),
            full((L,)),
        ],
        out_specs=full((G, L)),
        out_shape=jax.ShapeDtypeStruct((G, L), jnp.float32),
        scratch_shapes=[pltpu.VMEM((G, D1), jnp.float32)],
    )(w, features, pos_edges, W1, b1, Wc1, bc1, Wmu, bmu, Wclf, bclf)


def kernel(features, edges, pos_edges, W1, b1, Wf1, bf1, Wf2, bf2,
           Wc1, bc1, Wmu, bmu, Wls, bls, Wclf, bclf):
    w = _sc_edge_weights(edges)
    return _tc_head(w, features, pos_edges, W1, b1, Wc1, bc1, Wmu, bmu,
                    Wclf, bclf)


# speedup vs baseline: 419.2995x; 1.0048x over previous
import functools

import jax
import jax.numpy as jnp
from jax import lax
from jax.experimental import pallas as pl
from jax.experimental.pallas import tpu as pltpu
from jax.experimental.pallas import tpu_sc as plsc

G = 32
N = 2048
E = 16384
D = 128
PE = 128
NC, NS = 2, 16
LANES = 16


def _sc_edge_weights(edges):
    mesh = plsc.VectorSubcoreMesh(core_axis_name="c", subcore_axis_name="s")

    @functools.partial(
        pl.kernel,
        out_type=jax.ShapeDtypeStruct((G, N), jnp.float32),
        mesh=mesh,
        compiler_params=pltpu.CompilerParams(needs_layout_passes=False),
        scratch_types=[
            pltpu.VMEM((2, E), jnp.int32),
            pltpu.VMEM((LANES * N,), jnp.float32),
            pltpu.VMEM((LANES * N,), jnp.float32),
            pltpu.VMEM((N,), jnp.float32),
            pltpu.VMEM((N,), jnp.float32),
            pltpu.VMEM((N,), jnp.float32),
            pltpu.SemaphoreType.DMA,
        ],
    )
    def k(edges_hbm, w_hbm, edges_v, hist_v, hist2_v, dinv_v, cnt_v, w_v, sem):
        g = lax.axis_index("s") * NC + lax.axis_index("c")
        cp = pltpu.async_copy(edges_hbm.at[g], edges_v, sem)
        lane_off = lax.iota(jnp.int32, 16) * N
        zeros16 = jnp.zeros((16,), jnp.float32)
        ones16 = jnp.ones((16,), jnp.float32)

        def zero_body(j, carry):
            for l in range(LANES):
                hist_v[pl.ds(l * N + j * 16, 16)] = zeros16
                hist2_v[pl.ds(l * N + j * 16, 16)] = zeros16
            return carry

        lax.fori_loop(0, N // 16, zero_body, 0)
        cp.wait()

        def hist_body(i, carry):
            for u in range(8):
                c = edges_v[1, pl.ds(i * 128 + u * 16, 16)]
                plsc.addupdate_scatter(hist_v if u % 2 == 0 else hist2_v,
                                       [lane_off + c], ones16)
            return carry

        lax.fori_loop(0, E // 128, hist_body, 0)

        def _lane_reduce(j):
            vals = [hist_v[pl.ds(l * N + j * 16, 16)] for l in range(LANES)]
            vals += [hist2_v[pl.ds(l * N + j * 16, 16)] for l in range(LANES)]
            while len(vals) > 1:
                vals = [vals[i] + vals[i + 1] for i in range(0, len(vals), 2)]
            return vals[0]

        def dinv_body(j, carry):
            acc = _lane_reduce(j)
            cnt_v[pl.ds(j * 16, 16)] = acc
            deg = acc + 1.0
            bits = plsc.bitcast(deg, jnp.int32)
            y = plsc.bitcast(jnp.int32(0x5F3759DF) - lax.shift_right_logical(bits, 1),
                             jnp.float32)
            y = y * (1.5 - 0.5 * deg * y * y)
            y = y * (1.5 - 0.5 * deg * y * y)
            dinv_v[pl.ds(j * 16, 16)] = y
            return carry

        lax.fori_loop(0, N // 16, dinv_body, 0)

        def s_body(i, carry):
            for u in range(8):
                sl = pl.ds(i * 128 + u * 16, 16)
                r = edges_v[0, sl]
                c = edges_v[1, sl]
                dc = plsc.load_gather(dinv_v, [c])
                plsc.addupdate_scatter(hist_v if u % 2 == 0 else hist2_v,
                                       [lane_off + r], dc)
            return carry

        lax.fori_loop(0, E // 128, s_body, 0)

        def w_body(j, carry):
            s = _lane_reduce(j) - cnt_v[pl.ds(j * 16, 16)]
            dv = dinv_v[pl.ds(j * 16, 16)]
            w_v[pl.ds(j * 16, 16)] = dv * (s + dv)
            return carry

        lax.fori_loop(0, N // 16, w_body, 0)

        pltpu.sync_copy(w_v, w_hbm.at[g])

    return k(edges)


def _tc_head(w, features, pos_edges, W1, b1, Wc1, bc1, Wmu, bmu, Wclf, bclf):
    D1 = W1.shape[1]
    H = Wmu.shape[1]
    L = Wclf.shape[1]

    B = 8

    def body(w_ref, x_ref, pe_ref, W1_ref, b1_ref, Wc1_ref, bc1_ref,
             Wmu_ref, bmu_ref, Wclf_ref, bclf_ref, out_ref, v_acc):
        g = pl.program_id(0)

        @pl.when(g == 0)
        def _():
            v_acc[...] = jnp.zeros((G, D1), jnp.float32)

        wv = w_ref[...]
        rows = lax.broadcasted_iota(jnp.int32, (G, N), 0)
        acc = jnp.zeros((G, D1), jnp.float32)
        for b in range(B):
            gg = g * B + b
            P = jnp.where(rows == gg, wv, 0.0)
            acc = acc + jnp.dot(P, x_ref[b], preferred_element_type=jnp.float32)
        v_acc[...] += acc

        @pl.when(g == G // B - 1)
        def _():
            emb = (jnp.dot(v_acc[...], W1_ref[...],
                           preferred_element_type=jnp.float32)
                   + N * b1_ref[...]) * (1.0 / 16.0)
            pe = pe_ref[...]
            gids = lax.broadcasted_iota(jnp.int32, (PE, G), 1)
            oh_r = (pe[0][:, None] == gids).astype(jnp.float32)
            oh_c = (pe[1][:, None] == gids).astype(jnp.float32)
            A = lax.dot_general(oh_c, oh_r, (((0,), (0,)), ((), ())),
                                preferred_element_type=jnp.float32)
            eye = (lax.broadcasted_iota(jnp.int32, (G, G), 0)
                   == lax.broadcasted_iota(jnp.int32, (G, G), 1)
                   ).astype(jnp.float32)
            A1 = A + eye
            deg = jnp.sum(A1, axis=1)
            dinv = lax.rsqrt(deg)
            Ahat = dinv[:, None] * A1 * dinv[None, :]

            def mm(a, b):
                return jnp.dot(a, b, preferred_element_type=jnp.float32)

            h1 = jnp.maximum(mm(Ahat, mm(emb, Wc1_ref[...])) + bc1_ref[...], 0.0)
            mu = mm(Ahat, mm(h1, Wmu_ref[...])) + bmu_ref[...]
            logits = mm(mu, Wclf_ref[...]) + bclf_ref[...]
            m = jnp.max(logits, axis=1, keepdims=True)
            lse = m + jnp.log(jnp.sum(jnp.exp(logits - m), axis=1, keepdims=True))
            out_ref[...] = logits - lse

    full = lambda shape: pl.BlockSpec(shape, lambda g: tuple(0 for _ in shape))
    return pl.pallas_call(
        body,
        grid=(G // B,),
        in_specs=[
            full((G, N)),
            pl.BlockSpec((B, N, D), lambda g: (g, 0, 0)),
            full((2, PE)),
            full((D, D1)),
            full((D1,)),
            full((D1, Wc1.shape[1])),
            full((Wc1.shape[1],)),
            full((2 * H, H)),
            full((H,)),
            full((H, L)),
            full((L,)),
        ],
        out_specs=full((G, L)),
        out_shape=jax.ShapeDtypeStruct((G, L), jnp.float32),
        scratch_shapes=[pltpu.VMEM((G, D1), jnp.float32)],
    )(w, features, pos_edges, W1, b1, Wc1, bc1, Wmu, bmu, Wclf, bclf)


def kernel(features, edges, pos_edges, W1, b1, Wf1, bf1, Wf2, bf2,
           Wc1, bc1, Wmu, bmu, Wls, bls, Wclf, bclf):
    w = _sc_edge_weights(edges)
    return _tc_head(w, features, pos_edges, W1, b1, Wc1, bc1, Wmu, bmu,
                    Wclf, bclf)


# speedup vs baseline: 424.0436x; 1.0113x over previous
import functools

import jax
import jax.numpy as jnp
from jax import lax
from jax.experimental import pallas as pl
from jax.experimental.pallas import tpu as pltpu
from jax.experimental.pallas import tpu_sc as plsc

G = 32
N = 2048
E = 16384
D = 128
PE = 128
NC, NS = 2, 16
LANES = 16


def _sc_edge_weights(edges):
    mesh = plsc.VectorSubcoreMesh(core_axis_name="c", subcore_axis_name="s")

    @functools.partial(
        pl.kernel,
        out_type=jax.ShapeDtypeStruct((G, N), jnp.float32),
        mesh=mesh,
        compiler_params=pltpu.CompilerParams(needs_layout_passes=False),
        scratch_types=[
            pltpu.VMEM((2, E), jnp.int32),
            pltpu.VMEM((LANES * N,), jnp.float32),
            pltpu.VMEM((LANES * N,), jnp.float32),
            pltpu.VMEM((N,), jnp.float32),
            pltpu.VMEM((N,), jnp.float32),
            pltpu.VMEM((N,), jnp.float32),
            pltpu.SemaphoreType.DMA,
        ],
    )
    def k(edges_hbm, w_hbm, edges_v, hist_v, hist2_v, dinv_v, cnt_v, w_v, sem):
        g = lax.axis_index("s") * NC + lax.axis_index("c")
        cp = pltpu.async_copy(edges_hbm.at[g], edges_v, sem)
        lane_off = lax.iota(jnp.int32, 16) * N
        zeros16 = jnp.zeros((16,), jnp.float32)
        ones16 = jnp.ones((16,), jnp.float32)

        def zero_body(j, carry):
            for l in range(LANES):
                hist_v[pl.ds(l * N + j * 16, 16)] = zeros16
                hist2_v[pl.ds(l * N + j * 16, 16)] = zeros16
            return carry

        lax.fori_loop(0, N // 16, zero_body, 0)
        cp.wait()

        def hist_body(i, carry):
            for u in range(16):
                c = edges_v[1, pl.ds(i * 256 + u * 16, 16)]
                plsc.addupdate_scatter(hist_v if u % 2 == 0 else hist2_v,
                                       [lane_off + c], ones16)
            return carry

        lax.fori_loop(0, E // 256, hist_body, 0)

        def _lane_reduce(j):
            vals = [hist_v[pl.ds(l * N + j * 16, 16)] for l in range(LANES)]
            vals += [hist2_v[pl.ds(l * N + j * 16, 16)] for l in range(LANES)]
            while len(vals) > 1:
                vals = [vals[i] + vals[i + 1] for i in range(0, len(vals), 2)]
            return vals[0]

        def dinv_body(j, carry):
            acc = _lane_reduce(j)
            cnt_v[pl.ds(j * 16, 16)] = acc
            deg = acc + 1.0
            bits = plsc.bitcast(deg, jnp.int32)
            y = plsc.bitcast(jnp.int32(0x5F3759DF) - lax.shift_right_logical(bits, 1),
                             jnp.float32)
            y = y * (1.5 - 0.5 * deg * y * y)
            y = y * (1.5 - 0.5 * deg * y * y)
            dinv_v[pl.ds(j * 16, 16)] = y
            return carry

        lax.fori_loop(0, N // 16, dinv_body, 0)

        def s_body(i, carry):
            for u in range(16):
                sl = pl.ds(i * 256 + u * 16, 16)
                r = edges_v[0, sl]
                c = edges_v[1, sl]
                dc = plsc.load_gather(dinv_v, [c])
                plsc.addupdate_scatter(hist_v if u % 2 == 0 else hist2_v,
                                       [lane_off + r], dc)
            return carry

        lax.fori_loop(0, E // 256, s_body, 0)

        def w_body(j, carry):
            s = _lane_reduce(j) - cnt_v[pl.ds(j * 16, 16)]
            dv = dinv_v[pl.ds(j * 16, 16)]
            w_v[pl.ds(j * 16, 16)] = dv * (s + dv)
            return carry

        lax.fori_loop(0, N // 16, w_body, 0)

        pltpu.sync_copy(w_v, w_hbm.at[g])

    return k(edges)


def _tc_head(w, features, pos_edges, W1, b1, Wc1, bc1, Wmu, bmu, Wclf, bclf):
    D1 = W1.shape[1]
    H = Wmu.shape[1]
    L = Wclf.shape[1]

    B = 8

    def body(w_ref, x_ref, pe_ref, W1_ref, b1_ref, Wc1_ref, bc1_ref,
             Wmu_ref, bmu_ref, Wclf_ref, bclf_ref, out_ref, v_acc):
        g = pl.program_id(0)

        @pl.when(g == 0)
        def _():
            v_acc[...] = jnp.zeros((G, D1), jnp.float32)

        wv = w_ref[...]
        rows = lax.broadcasted_iota(jnp.int32, (G, N), 0)
        acc = jnp.zeros((G, D1), jnp.float32)
        for b in range(B):
            gg = g * B + b
            P = jnp.where(rows == gg, wv, 0.0)
            acc = acc + jnp.dot(P, x_ref[b], preferred_element_type=jnp.float32)
        v_acc[...] += acc

        @pl.when(g == G // B - 1)
        def _():
            emb = (jnp.dot(v_acc[...], W1_ref[...],
                           preferred_element_type=jnp.float32)
                   + N * b1_ref[...]) * (1.0 / 16.0)
            pe = pe_ref[...]
            gids = lax.broadcasted_iota(jnp.int32, (PE, G), 1)
            oh_r = (pe[0][:, None] == gids).astype(jnp.float32)
            oh_c = (pe[1][:, None] == gids).astype(jnp.float32)
            A = lax.dot_general(oh_c, oh_r, (((0,), (0,)), ((), ())),
                                preferred_element_type=jnp.float32)
            eye = (lax.broadcasted_iota(jnp.int32, (G, G), 0)
                   == lax.broadcasted_iota(jnp.int32, (G, G), 1)
                   ).astype(jnp.float32)
            A1 = A + eye
            deg = jnp.sum(A1, axis=1)
            dinv = lax.rsqrt(deg)
            Ahat = dinv[:, None] * A1 * dinv[None, :]

            def mm(a, b):
                return jnp.dot(a, b, preferred_element_type=jnp.float32)

            h1 = jnp.maximum(mm(Ahat, mm(emb, Wc1_ref[...])) + bc1_ref[...], 0.0)
            mu = mm(Ahat, mm(h1, Wmu_ref[...])) + bmu_ref[...]
            logits = mm(mu, Wclf_ref[...]) + bclf_ref[...]
            m = jnp.max(logits, axis=1, keepdims=True)
            lse = m + jnp.log(jnp.sum(jnp.exp(logits - m), axis=1, keepdims=True))
            out_ref[...] = logits - lse

    full = lambda shape: pl.BlockSpec(shape, lambda g: tuple(0 for _ in shape))
    return pl.pallas_call(
        body,
        grid=(G // B,),
        in_specs=[
            full((G, N)),
            pl.BlockSpec((B, N, D), lambda g: (g, 0, 0)),
            full((2, PE)),
            full((D, D1)),
            full((D1,)),
            full((D1, Wc1.shape[1])),
            full((Wc1.shape[1],)),
            full((2 * H, H)),
            full((H,)),
            full((H, L)),
            full((L,)),
        ],
        out_specs=full((G, L)),
        out_shape=jax.ShapeDtypeStruct((G, L), jnp.float32),
        scratch_shapes=[pltpu.VMEM((G, D1), jnp.float32)],
    )(w, features, pos_edges, W1, b1, Wc1, bc1, Wmu, bmu, Wclf, bclf)


def kernel(features, edges, pos_edges, W1, b1, Wf1, bf1, Wf2, bf2,
           Wc1, bc1, Wmu, bmu, Wls, bls, Wclf, bclf):
    w = _sc_edge_weights(edges)
    return _tc_head(w, features, pos_edges, W1, b1, Wc1, bc1, Wmu, bmu,
                    Wclf, bclf)
